# Initial kernel scaffold; baseline (speedup 1.0000x reference)
import jax
import jax.numpy as jnp
from jax.experimental import pallas as pl


def kernel(x_user, x_wallet, edge_index_uw, edge_index_wu, edge_index_uu, Wl_u2w, bl_u2w, Wr_u2w, Wl_w2u, bl_w2u, Wr_w2u, Wl_u2u, bl_u2u, Wr_u2u, g_user, b_user, g_wallet, b_wallet, W_gat, att_src, att_dst, b_gat, g2, b2, Wp, bp):
    raise NotImplementedError("write your pallas kernel here")


# speedup vs baseline: 3.8905x; 3.8905x over previous
import functools

import jax
import jax.numpy as jnp
from jax.experimental import pallas as pl

_HID = 128
_H = 4
_C = 32
_BLK = 2000


def _dot(a, b):
    return jnp.dot(a, b, preferred_element_type=jnp.float32)


def _k_sage(s_wu, cnt_wu, s_uu, cnt_uu, xu, Wl1, bl1, Wr1, Wl2, bl2, Wr2,
            hpre, psum, psumsq):
    i = pl.program_id(0)
    mwu = s_wu[:] / jnp.maximum(cnt_wu[:], 1.0)
    muu = s_uu[:] / jnp.maximum(cnt_uu[:], 1.0)
    a = jnp.maximum(_dot(mwu, Wl1[:]) + bl1[:] + _dot(xu[:], Wr1[:]), 0.0)
    b = jnp.maximum(_dot(muu, Wl2[:]) + bl2[:] + _dot(xu[:], Wr2[:]), 0.0)
    h = a + b
    hpre[:] = h

    @pl.when(i == 0)
    def _():
        psum[:] = jnp.zeros_like(psum)
        psumsq[:] = jnp.zeros_like(psumsq)

    psum[:] += jnp.sum(h, axis=0, keepdims=True)
    psumsq[:] += jnp.sum(h * h, axis=0, keepdims=True)


def _k_bn_gat(hpre, scale, shift, Wg, As, Ad, hmat, asrc, adst):
    hu = hpre[:] * scale[:] + shift[:]
    hm = _dot(hu, Wg[:])
    hmat[:] = hm
    asrc[:] = _dot(hm, As[:])
    adst[:] = _dot(hm, Ad[:])


def _k_relu_stats(gat, bg, h2, psum, psumsq):
    i = pl.program_id(0)
    h = jnp.maximum(gat[:] + bg[:], 0.0)
    h2[:] = h

    @pl.when(i == 0)
    def _():
        psum[:] = jnp.zeros_like(psum)
        psumsq[:] = jnp.zeros_like(psumsq)

    psum[:] += jnp.sum(h, axis=0, keepdims=True)
    psumsq[:] += jnp.sum(h * h, axis=0, keepdims=True)


def _k_bn_proj(h2, scale, shift, Wp, bp, out):
    hn = h2[:] * scale[:] + shift[:]
    out[:] = jnp.maximum(_dot(hn, Wp[:]) + bp[:], 0.0)


def _row_spec(width):
    return pl.BlockSpec((_BLK, width), lambda i: (i, 0))


def _full_spec(shape):
    return pl.BlockSpec(shape, lambda i: tuple(0 for _ in shape))


def _bn_coeffs(psum, psumsq, n, g, b):
    mu = psum[0] / n
    var = psumsq[0] / n - mu * mu
    scale = g / jnp.sqrt(var + 1e-5)
    shift = b - mu * scale
    return scale[None, :], shift[None, :]


@functools.partial(jax.jit, static_argnums=())
def kernel(x_user, x_wallet, edge_index_uw, edge_index_wu, edge_index_uu,
           Wl_u2w, bl_u2w, Wr_u2w, Wl_w2u, bl_w2u, Wr_w2u, Wl_u2u, bl_u2u,
           Wr_u2u, g_user, b_user, g_wallet, b_wallet, W_gat, att_src,
           att_dst, b_gat, g2, b2, Wp, bp):
    n = x_user.shape[0]
    dw = x_wallet.shape[1]
    grid = (n // _BLK,)
    f32 = jnp.float32

    src_w, dst_w = edge_index_wu[0], edge_index_wu[1]
    src_u, dst_u = edge_index_uu[0], edge_index_uu[1]
    e_w = src_w.shape[0]
    e_u = src_u.shape[0]
    s_wu = jax.ops.segment_sum(x_wallet[src_w], dst_w, num_segments=n)
    cnt_wu = jax.ops.segment_sum(jnp.ones((e_w, 1), f32), dst_w, num_segments=n)
    s_uu = jax.ops.segment_sum(x_user[src_u], dst_u, num_segments=n)
    cnt_uu = jax.ops.segment_sum(jnp.ones((e_u, 1), f32), dst_u, num_segments=n)

    hpre, ps1, pq1 = pl.pallas_call(
        _k_sage,
        grid=grid,
        in_specs=[
            _row_spec(dw), _row_spec(1), _row_spec(_HID), _row_spec(1),
            _row_spec(_HID),
            _full_spec((dw, _HID)), _full_spec((1, _HID)),
            _full_spec((_HID, _HID)),
            _full_spec((_HID, _HID)), _full_spec((1, _HID)),
            _full_spec((_HID, _HID)),
        ],
        out_specs=[
            _row_spec(_HID),
            _full_spec((1, _HID)), _full_spec((1, _HID)),
        ],
        out_shape=[
            jax.ShapeDtypeStruct((n, _HID), f32),
            jax.ShapeDtypeStruct((1, _HID), f32),
            jax.ShapeDtypeStruct((1, _HID), f32),
        ],
    )(s_wu, cnt_wu, s_uu, cnt_uu, x_user,
      Wl_w2u, bl_w2u[None, :], Wr_w2u,
      Wl_u2u, bl_u2u[None, :], Wr_u2u)

    scale1, shift1 = _bn_coeffs(ps1, pq1, n, g_user, b_user)

    repmask = jnp.repeat(jnp.eye(_H, dtype=f32), _C, axis=0)
    As = att_src.reshape(_H * _C)[:, None] * repmask
    Ad = att_dst.reshape(_H * _C)[:, None] * repmask

    hmat, asrc, adst = pl.pallas_call(
        _k_bn_gat,
        grid=grid,
        in_specs=[
            _row_spec(_HID), _full_spec((1, _HID)), _full_spec((1, _HID)),
            _full_spec((_HID, _HID)), _full_spec((_HID, _H)),
            _full_spec((_HID, _H)),
        ],
        out_specs=[_row_spec(_HID), _row_spec(_H), _row_spec(_H)],
        out_shape=[
            jax.ShapeDtypeStruct((n, _HID), f32),
            jax.ShapeDtypeStruct((n, _H), f32),
            jax.ShapeDtypeStruct((n, _H), f32),
        ],
    )(hpre, scale1, shift1, W_gat, As, Ad)

    loop = jnp.arange(n, dtype=edge_index_uu.dtype)
    src = jnp.concatenate([src_u, loop])
    dst = jnp.concatenate([dst_u, loop])
    e = jax.nn.leaky_relu(asrc[src] + adst[dst], 0.2)
    emax = jax.ops.segment_max(e, dst, num_segments=n)
    emax = jnp.where(jnp.isfinite(emax), emax, 0.0)
    ex = jnp.exp(e - emax[dst])
    den = jax.ops.segment_sum(ex, dst, num_segments=n)
    alpha = ex / (den[dst] + 1e-16)
    msg = hmat[src].reshape(-1, _H, _C) * alpha[:, :, None]
    gat = jax.ops.segment_sum(msg.reshape(-1, _H * _C), dst, num_segments=n)

    h2, ps2, pq2 = pl.pallas_call(
        _k_relu_stats,
        grid=grid,
        in_specs=[_row_spec(_HID), _full_spec((1, _HID))],
        out_specs=[
            _row_spec(_HID),
            _full_spec((1, _HID)), _full_spec((1, _HID)),
        ],
        out_shape=[
            jax.ShapeDtypeStruct((n, _HID), f32),
            jax.ShapeDtypeStruct((1, _HID), f32),
            jax.ShapeDtypeStruct((1, _HID), f32),
        ],
    )(gat, b_gat[None, :])

    scale2, shift2 = _bn_coeffs(ps2, pq2, n, g2, b2)

    out_d = Wp.shape[1]
    out = pl.pallas_call(
        _k_bn_proj,
        grid=grid,
        in_specs=[
            _row_spec(_HID), _full_spec((1, _HID)), _full_spec((1, _HID)),
            _full_spec((_HID, out_d)), _full_spec((1, out_d)),
        ],
        out_specs=_row_spec(out_d),
        out_shape=jax.ShapeDtypeStruct((n, out_d), f32),
    )(h2, scale2, shift2, Wp, bp[None, :])

    return out
